# baseline (speedup 1.0000x reference)
import functools

import jax
import jax.numpy as jnp
from jax import lax
from jax.experimental import pallas as pl
from jax.experimental.pallas import tpu as pltpu
from jax.experimental.pallas import tpu_sc as plsc

_NUM_SAMPLES = 128
_NEAR = 0.1
_FAR = 100.0
_STEP = (_FAR - _NEAR) / (_NUM_SAMPLES - 1)
_BN = 4096
_BUF = 256


def _tc_body(o_ref, d_ref, pos_ref, view_ref):
    o = jnp.transpose(o_ref[:])
    d = jnp.transpose(d_ref[:])
    dn = d / (jnp.sqrt(jnp.sum(d * d, axis=1, keepdims=True)) + 1e-8)
    lane = jax.lax.broadcasted_iota(jnp.int32, (1, _NUM_SAMPLES), 1)
    depth_row = _NEAR + lane.astype(jnp.float32) * _STEP
    for c in range(3):
        oc = o[:, c : c + 1]
        dc = dn[:, c : c + 1]
        pos_ref[c] = oc + dc * depth_row
        view_ref[c] = jnp.broadcast_to(dc, (_BN, _NUM_SAMPLES))


def _sc_body(n_rows, dep_hbm, del_hbm, dep_buf, del_buf):
    info = plsc.get_sparse_core_info()
    nw = info.num_cores * info.num_subcores
    wid = lax.axis_index("s") * info.num_cores + lax.axis_index("c")
    rows = n_rows // nw
    dep_vals = []
    for j in range(_NUM_SAMPLES // 16):
        idx = lax.iota(jnp.int32, 16) + 16 * j
        dep_vals.append(_NEAR + idx.astype(jnp.float32) * _STEP)
    del_val = jnp.full((16,), _STEP, jnp.float32)

    def _fill(i, carry):
        for j in range(_NUM_SAMPLES // 16):
            dep_buf[i, pl.ds(16 * j, 16)] = dep_vals[j]
            del_buf[i, pl.ds(16 * j, 16)] = del_val
        return carry

    lax.fori_loop(0, _BUF, _fill, 0)
    base = wid * rows
    for t in range(rows // _BUF):
        pltpu.sync_copy(dep_buf, dep_hbm.at[pl.ds(base + t * _BUF, _BUF)])
        pltpu.sync_copy(del_buf, del_hbm.at[pl.ds(base + t * _BUF, _BUF)])


@jax.jit
def kernel(origins, directions):
    n, _ = origins.shape
    s = _NUM_SAMPLES
    grid = (n // _BN,)
    pos3, view3 = pl.pallas_call(
        _tc_body,
        grid=grid,
        in_specs=[
            pl.BlockSpec((3, _BN), lambda i: (0, i)),
            pl.BlockSpec((3, _BN), lambda i: (0, i)),
        ],
        out_specs=[
            pl.BlockSpec((3, _BN, s), lambda i: (0, i, 0)),
            pl.BlockSpec((3, _BN, s), lambda i: (0, i, 0)),
        ],
        out_shape=[
            jax.ShapeDtypeStruct((3, n, s), jnp.float32),
            jax.ShapeDtypeStruct((3, n, s), jnp.float32),
        ],
    )(origins.T, directions.T)
    sc_const = functools.partial(
        pl.kernel,
        out_type=[
            jax.ShapeDtypeStruct((n, s), jnp.float32),
            jax.ShapeDtypeStruct((n, s), jnp.float32),
        ],
        mesh=plsc.VectorSubcoreMesh(core_axis_name="c", subcore_axis_name="s"),
        scratch_types=[
            pltpu.VMEM((_BUF, s), jnp.float32),
            pltpu.VMEM((_BUF, s), jnp.float32),
        ],
    )(functools.partial(_sc_body, n))
    depths, deltas = sc_const()
    positions = pos3.transpose(1, 2, 0)
    view_directions = view3.transpose(1, 2, 0)
    return positions, view_directions, depths, deltas
